# baseline (speedup 1.0000x reference)
import functools

import numpy as np
import jax
import jax.numpy as jnp
from jax import lax
from jax.experimental import pallas as pl
from jax.experimental.pallas import tpu as pltpu
from jax.experimental.pallas import tpu_sc as plsc

_CW = 8192
_D = 128
_NF = _D // 2
_NC = 2
_NS = 16
_NW = _NC * _NS
_BPW = _CW // _NW
_CHUNK = 32
_NCHUNK = _BPW // _CHUNK
_NK = 8
_KSPAN = _BPW // _NK
_LANES = 16
_FCH = _NF // _LANES


def _freqs() -> np.ndarray:
    return np.power(10000.0, -np.arange(0, _D, 2, dtype=np.float64) / _D)


def _make_seeds() -> np.ndarray:
    th = _freqs()
    rows = (np.arange(_NW * _NK) * _KSPAN)[:, None]
    ang = rows * th[None, :]
    out = np.empty((_NW * _NK, _D), dtype=np.float64)
    out[:, 0::2] = np.sin(ang)
    out[:, 1::2] = np.cos(ang)
    return out.reshape(-1).astype(np.float32)


def _make_trig() -> np.ndarray:
    th = _freqs()
    a = np.repeat(np.cos(th), 2)
    b = np.empty(_D, dtype=np.float64)
    b[0::2] = np.sin(th)
    b[1::2] = -np.sin(th)
    return np.concatenate([a, b]).astype(np.float32)


_CONST = np.concatenate([_make_seeds(), _make_trig()])
_TRIG_OFF = _NW * _NK * _D

_mesh = plsc.VectorSubcoreMesh(core_axis_name="c", subcore_axis_name="s")


@functools.partial(
    pl.kernel,
    out_type=jax.ShapeDtypeStruct((_CW, _D), jnp.float32),
    mesh=_mesh,
    scratch_types=[
        pltpu.VMEM((_BPW,), jnp.int32),
        pltpu.VMEM((_NK * _D,), jnp.float32),
        pltpu.VMEM((2 * _D,), jnp.float32),
        pltpu.VMEM((_BPW, _D), jnp.float32),
        pltpu.SemaphoreType.DMA,
        [pltpu.SemaphoreType.DMA] * _NCHUNK,
        pltpu.SemaphoreType.DMA,
    ],
)
def _embed_sc(x_hbm, const_hbm, w_hbm, out_hbm,
              idx_v, seed_v, trig_v, rows_v, psem, gsems, ssem):
    wid = lax.axis_index("s") * _NC + lax.axis_index("c")
    base = wid * _BPW

    pre = [
        pltpu.async_copy(x_hbm.at[pl.ds(base, _BPW)], idx_v, psem),
        pltpu.async_copy(const_hbm.at[pl.ds(wid * _NK * _D, _NK * _D)],
                         seed_v, psem),
        pltpu.async_copy(const_hbm.at[pl.ds(_TRIG_OFF, 2 * _D)],
                         trig_v, psem),
    ]
    for p in pre:
        p.wait()

    swap = lax.iota(jnp.int32, _LANES) ^ 1
    nj = _D // _LANES
    rot_a = [trig_v[pl.ds(j * _LANES, _LANES)] for j in range(nj)]
    rot_b = [trig_v[pl.ds(_D + j * _LANES, _LANES)] for j in range(nj)]

    kpc = _CHUNK // _KSPAN if _CHUNK >= _KSPAN else 1
    gathers = []
    for g in range(_NCHUNK):
        state = []
        for kk in range(kpc):
            off = ((g * _CHUNK) // _KSPAN + kk) * _D
            for j in range(nj):
                state.append(seed_v[pl.ds(off + j * _LANES, _LANES)])

        def step(i, st):
            new = []
            for kk in range(kpc):
                row = g * _CHUNK + kk * _KSPAN + i
                for j in range(nj):
                    v = st[kk * nj + j]
                    rows_v[row, pl.ds(j * _LANES, _LANES)] = v
                    new.append(v * rot_a[j] + v[swap] * rot_b[j])
            return tuple(new)

        lax.fori_loop(0, min(_CHUNK, _KSPAN), step, tuple(state))

        row_sl = pl.ds(g * _CHUNK, _CHUNK)
        gathers.append(pltpu.async_copy(
            w_hbm.at[idx_v.at[row_sl]], rows_v.at[row_sl, :], gsems[g],
            add=True))

    stores = []
    for g in range(_NCHUNK):
        row_sl = pl.ds(g * _CHUNK, _CHUNK)
        gathers[g].wait()
        stores.append(pltpu.async_copy(
            rows_v.at[row_sl, :],
            out_hbm.at[pl.ds(base + g * _CHUNK, _CHUNK)], ssem))
    for s in stores:
        s.wait()


def kernel(x, embed_weight):
    return _embed_sc(x, jnp.asarray(_CONST), embed_weight)


# speedup vs baseline: 1.0082x; 1.0082x over previous
import functools

import numpy as np
import jax
import jax.numpy as jnp
from jax import lax
from jax.experimental import pallas as pl
from jax.experimental.pallas import tpu as pltpu
from jax.experimental.pallas import tpu_sc as plsc

_CW = 8192
_D = 128
_NF = _D // 2
_NC = 2
_NS = 16
_NW = _NC * _NS
_BPW = _CW // _NW
_CHUNK = 64
_NCHUNK = _BPW // _CHUNK
_NK = 4
_KSPAN = _BPW // _NK
_LANES = 16
_FCH = _NF // _LANES


def _freqs() -> np.ndarray:
    return np.power(10000.0, -np.arange(0, _D, 2, dtype=np.float64) / _D)


def _make_seeds() -> np.ndarray:
    th = _freqs()
    rows = (np.arange(_NW * _NK) * _KSPAN)[:, None]
    ang = rows * th[None, :]
    out = np.empty((_NW * _NK, _D), dtype=np.float64)
    out[:, 0::2] = np.sin(ang)
    out[:, 1::2] = np.cos(ang)
    return out.reshape(-1).astype(np.float32)


def _make_trig() -> np.ndarray:
    th = _freqs()
    a = np.repeat(np.cos(th), 2)
    b = np.empty(_D, dtype=np.float64)
    b[0::2] = np.sin(th)
    b[1::2] = -np.sin(th)
    return np.concatenate([a, b]).astype(np.float32)


_CONST = np.concatenate([_make_seeds(), _make_trig()])
_TRIG_OFF = _NW * _NK * _D

_mesh = plsc.VectorSubcoreMesh(core_axis_name="c", subcore_axis_name="s")


@functools.partial(
    pl.kernel,
    out_type=jax.ShapeDtypeStruct((_CW, _D), jnp.float32),
    mesh=_mesh,
    scratch_types=[
        pltpu.VMEM((_BPW,), jnp.int32),
        pltpu.VMEM((_NK * _D,), jnp.float32),
        pltpu.VMEM((2 * _D,), jnp.float32),
        pltpu.VMEM((_BPW, _D), jnp.float32),
        pltpu.SemaphoreType.DMA,
        pltpu.SemaphoreType.DMA,
        [pltpu.SemaphoreType.DMA] * _NCHUNK,
        pltpu.SemaphoreType.DMA,
    ],
)
def _embed_sc(x_hbm, const_hbm, w_hbm, out_hbm,
              idx_v, seed_v, trig_v, rows_v, isem, psem, gsems, ssem):
    wid = lax.axis_index("s") * _NC + lax.axis_index("c")
    base = wid * _BPW

    idx_cp = pltpu.async_copy(x_hbm.at[pl.ds(base, _BPW)], idx_v, isem)
    pre = [
        pltpu.async_copy(const_hbm.at[pl.ds(wid * _NK * _D, _NK * _D)],
                         seed_v, psem),
        pltpu.async_copy(const_hbm.at[pl.ds(_TRIG_OFF, 2 * _D)],
                         trig_v, psem),
    ]
    for p in pre:
        p.wait()

    swap = lax.iota(jnp.int32, _LANES) ^ 1
    nj = _D // _LANES
    rot_a = [trig_v[pl.ds(j * _LANES, _LANES)] for j in range(nj)]
    rot_b = [trig_v[pl.ds(_D + j * _LANES, _LANES)] for j in range(nj)]

    kpc = _CHUNK // _KSPAN if _CHUNK >= _KSPAN else 1
    gathers = []
    for g in range(_NCHUNK):
        state = []
        for kk in range(kpc):
            off = ((g * _CHUNK) // _KSPAN + kk) * _D
            for j in range(nj):
                state.append(seed_v[pl.ds(off + j * _LANES, _LANES)])

        def step(i, st):
            new = []
            for kk in range(kpc):
                row = g * _CHUNK + kk * _KSPAN + i
                for j in range(nj):
                    v = st[kk * nj + j]
                    rows_v[row, pl.ds(j * _LANES, _LANES)] = v
                    new.append(v * rot_a[j] + v[swap] * rot_b[j])
            return tuple(new)

        lax.fori_loop(0, min(_CHUNK, _KSPAN), step, tuple(state))

        if g == 0:
            idx_cp.wait()
        row_sl = pl.ds(g * _CHUNK, _CHUNK)
        gathers.append(pltpu.async_copy(
            w_hbm.at[idx_v.at[row_sl]], rows_v.at[row_sl, :], gsems[g],
            add=True))

    stores = []
    for g in range(_NCHUNK):
        row_sl = pl.ds(g * _CHUNK, _CHUNK)
        gathers[g].wait()
        stores.append(pltpu.async_copy(
            rows_v.at[row_sl, :],
            out_hbm.at[pl.ds(base + g * _CHUNK, _CHUNK)], ssem))
    for s in stores:
        s.wait()


def kernel(x, embed_weight):
    return _embed_sc(x, jnp.asarray(_CONST), embed_weight)
